# speedup vs baseline: 1.2766x; 1.2766x over previous
import functools

import jax
import jax.numpy as jnp
from jax import lax
from jax.experimental import pallas as pl
from jax.experimental.pallas import tpu as pltpu
from jax.experimental.pallas import tpu_sc as plsc

_B = 16384
_ED = 256
_HD = 512
_CD = 256
_K = 8192
_BETA = 0.25

_BB = 256
_NBLK = _B // _BB

_NW = 32
_BPW = _B // _NW
_CHUNK = 128
_NCHUNK = _BPW // _CHUNK


def _enc_vq_body(subj_ref, rel_ref, obj_ref, w1_ref, b1_ref, w2_ref, b2_ref,
                 w3_ref, b3_ref, cb_ref, z_ref, codes_ref, cbsq_ref):
    @pl.when(pl.program_id(0) == 0)
    def _():
        cb = cb_ref[...]
        ones = jnp.ones((1, _CD), jnp.float32)
        cbsq_ref[...] = lax.dot_general(ones, cb * cb,
                                        (((1,), (1,)), ((), ())))

    pv = jnp.concatenate([subj_ref[...], rel_ref[...], obj_ref[...]], axis=1)
    h = jnp.maximum(jnp.dot(pv, w1_ref[...]) + b1_ref[...], 0.0)
    h = jnp.maximum(jnp.dot(h, w2_ref[...]) + b2_ref[...], 0.0)
    z = jnp.dot(h, w3_ref[...]) + b3_ref[...]
    z_ref[...] = z

    z_sq = jnp.sum(z * z, axis=1, keepdims=True)
    zc = lax.dot_general(z, cb_ref[...], (((1,), (1,)), ((), ())))
    dist = (z_sq + cbsq_ref[...]) - 2.0 * zc

    m = jnp.min(dist, axis=1, keepdims=True)
    ids = lax.broadcasted_iota(jnp.int32, dist.shape, 1)
    codes = jnp.min(jnp.where(dist == m, ids, jnp.int32(_K)), axis=1)
    codes_ref[0, 0, :] = codes


def _dec_body(zq_ref, z_ref, w1_ref, b1_ref, w2_ref, b2_ref, w3_ref, b3_ref,
              s_ref, r_ref, o_ref, sse_ref):
    zq = zq_ref[...]
    z = z_ref[...]
    d = zq - z
    part = jnp.sum(d * d)

    @pl.when(pl.program_id(0) == 0)
    def _():
        sse_ref[...] = jnp.zeros((1, 1), jnp.float32)

    sse_ref[...] += jnp.reshape(part, (1, 1))

    zq_st = z + d
    h = jnp.maximum(jnp.dot(zq_st, w1_ref[...]) + b1_ref[...], 0.0)
    h = jnp.maximum(jnp.dot(h, w2_ref[...]) + b2_ref[...], 0.0)
    p = jnp.dot(h, w3_ref[...]) + b3_ref[...]
    s_ref[...] = p[:, :_ED]
    r_ref[...] = p[:, _ED:2 * _ED]
    o_ref[...] = p[:, 2 * _ED:]


def _sc_gather_fn():
    mesh = plsc.VectorSubcoreMesh(core_axis_name="c", subcore_axis_name="s")

    @functools.partial(
        pl.kernel,
        mesh=mesh,
        out_type=jax.ShapeDtypeStruct((_B, _CD), jnp.float32),
        scratch_types=[
            pltpu.VMEM((_NCHUNK, _CHUNK), jnp.int32),
            pltpu.VMEM((_CHUNK, _CD), jnp.float32),
            pltpu.VMEM((_CHUNK, _CD), jnp.float32),
            pltpu.SemaphoreType.DMA,
            pltpu.SemaphoreType.DMA,
        ],
    )
    def gather(cb_hbm, codes_hbm, out_hbm, idx_v, buf0, buf1, sem0, sem1):
        wid = lax.axis_index("s") * 2 + lax.axis_index("c")
        base = wid * _BPW
        for c in range(_NCHUNK):
            pltpu.sync_copy(codes_hbm.at[pl.ds(base + c * _CHUNK, _CHUNK)],
                            idx_v.at[c])
        bufs = (buf0, buf1)
        sems = (sem0, sem1)
        for c in range(_NCHUNK):
            buf = bufs[c % 2]
            pltpu.async_copy(cb_hbm.at[idx_v.at[c]], buf, sems[c % 2]).wait()
            pltpu.sync_copy(buf, out_hbm.at[pl.ds(base + c * _CHUNK, _CHUNK)])

    return gather


def _enc_vq_call(subj, rel, obj, w1, b1, w2, b2, w3, b3, cb):
    return pl.pallas_call(
        _enc_vq_body,
        grid=(_NBLK,),
        in_specs=[
            pl.BlockSpec((_BB, _ED), lambda i: (i, 0)),
            pl.BlockSpec((_BB, _ED), lambda i: (i, 0)),
            pl.BlockSpec((_BB, _ED), lambda i: (i, 0)),
            pl.BlockSpec((3 * _ED, _HD), lambda i: (0, 0)),
            pl.BlockSpec((1, _HD), lambda i: (0, 0)),
            pl.BlockSpec((_HD, _HD), lambda i: (0, 0)),
            pl.BlockSpec((1, _HD), lambda i: (0, 0)),
            pl.BlockSpec((_HD, _CD), lambda i: (0, 0)),
            pl.BlockSpec((1, _CD), lambda i: (0, 0)),
            pl.BlockSpec((_K, _CD), lambda i: (0, 0)),
        ],
        out_specs=[
            pl.BlockSpec((_BB, _CD), lambda i: (i, 0)),
            pl.BlockSpec((1, 1, _BB), lambda i: (i, 0, 0)),
        ],
        out_shape=[
            jax.ShapeDtypeStruct((_B, _CD), jnp.float32),
            jax.ShapeDtypeStruct((_NBLK, 1, _BB), jnp.int32),
        ],
        scratch_shapes=[pltpu.VMEM((1, _K), jnp.float32)],
    )(subj, rel, obj, w1, b1, w2, b2, w3, b3, cb)


def _dec_call(zq, z, w1, b1, w2, b2, w3, b3):
    return pl.pallas_call(
        _dec_body,
        grid=(_NBLK,),
        in_specs=[
            pl.BlockSpec((_BB, _CD), lambda i: (i, 0)),
            pl.BlockSpec((_BB, _CD), lambda i: (i, 0)),
            pl.BlockSpec((_CD, _HD), lambda i: (0, 0)),
            pl.BlockSpec((1, _HD), lambda i: (0, 0)),
            pl.BlockSpec((_HD, _HD), lambda i: (0, 0)),
            pl.BlockSpec((1, _HD), lambda i: (0, 0)),
            pl.BlockSpec((_HD, 3 * _ED), lambda i: (0, 0)),
            pl.BlockSpec((1, 3 * _ED), lambda i: (0, 0)),
        ],
        out_specs=[
            pl.BlockSpec((_BB, _ED), lambda i: (i, 0)),
            pl.BlockSpec((_BB, _ED), lambda i: (i, 0)),
            pl.BlockSpec((_BB, _ED), lambda i: (i, 0)),
            pl.BlockSpec((1, 1), lambda i: (0, 0)),
        ],
        out_shape=[
            jax.ShapeDtypeStruct((_B, _ED), jnp.float32),
            jax.ShapeDtypeStruct((_B, _ED), jnp.float32),
            jax.ShapeDtypeStruct((_B, _ED), jnp.float32),
            jax.ShapeDtypeStruct((1, 1), jnp.float32),
        ],
    )(zq, z, w1, b1, w2, b2, w3, b3)


def kernel(subj_emb, rel_emb, obj_emb, enc_W1, enc_b1, enc_W2, enc_b2,
           enc_W3, enc_b3, codebook, dec_W1, dec_b1, dec_W2, dec_b2,
           dec_W3, dec_b3):
    z, codes3 = _enc_vq_call(
        subj_emb, rel_emb, obj_emb, enc_W1, enc_b1.reshape(1, _HD),
        enc_W2, enc_b2.reshape(1, _HD), enc_W3, enc_b3.reshape(1, _CD),
        codebook)
    codes = codes3.reshape(_B)

    z_q = _sc_gather_fn()(codebook, codes)

    subj_r, rel_r, obj_r, sse = _dec_call(
        z_q, z, dec_W1, dec_b1.reshape(1, _HD), dec_W2,
        dec_b2.reshape(1, _HD), dec_W3, dec_b3.reshape(1, 3 * _ED))

    codebook_loss = sse[0, 0] / jnp.float32(_B * _CD)
    commitment_loss = codebook_loss * jnp.float32(_BETA)
    total_vq = codebook_loss + commitment_loss
    return (subj_r, rel_r, obj_r, codes, codebook_loss, commitment_loss,
            total_vq)
